# baseline (speedup 1.0000x reference)
import jax
import jax.numpy as jnp
from jax.experimental import pallas as pl
from jax.experimental.pallas import tpu as pltpu

D_MODEL = 1024
N_STATIC = 64
N_TIME = 32
KEY_DIM = 128
MAX_BANK = 4096
B = 8
T = 2048

ROWS = 512
COLS = 256


def _gelu(x):
    return 0.5 * x * (1.0 + jax.lax.erf(x * 0.7071067811865476))


def _gate_prep_kernel(static_ref, bank_ref, phi_W1_ref, phi_b1_ref,
                      phi_W2_ref, phi_b2_ref, vp_W_ref, vp_b_ref,
                      rho_W1_ref, rho_b1_ref, rho_W2_ref, rho_b2_ref,
                      tp_W_ref, tp_b_ref, sp_W_ref, sp_b_ref,
                      lc_b1_ref, W1x_ref, W1c_ref,
                      alpha_ref, bias_ref, M_ref):
    static = static_ref[...]
    f32 = jnp.float32

    def dot(a, b):
        return jnp.dot(a, b, preferred_element_type=f32)

    @pl.when(pl.program_id(0) == 0)
    def _attention():
        def phi(x):
            h = _gelu(dot(x, phi_W1_ref[...]) + phi_b1_ref[...])
            return dot(h, phi_W2_ref[...]) + phi_b2_ref[...]

        static_tiled = jnp.broadcast_to(
            static[None],
            (MAX_BANK // B, B, N_STATIC)).reshape(MAX_BANK, N_STATIC)
        rows = jax.lax.broadcasted_iota(jnp.int32, (MAX_BANK, N_STATIC), 0)
        bank_upd = jnp.where(rows < B, static_tiled, bank_ref[...])

        q = phi(static)
        k = phi(bank_upd)
        v = dot(bank_upd, vp_W_ref[...]) + vp_b_ref[...]

        scores = jax.lax.dot_general(
            q, k, (((1,), (1,)), ((), ())),
            preferred_element_type=f32) / (KEY_DIM ** 0.5)
        weights = jax.nn.softmax(scores, axis=-1)
        context = dot(weights, v)

        gate_in = jnp.concatenate([q, context], axis=-1)
        g1 = _gelu(dot(gate_in, rho_W1_ref[...]) + rho_b1_ref[...])
        alpha = jax.nn.sigmoid(dot(g1, rho_W2_ref[...]) + rho_b2_ref[...])
        alpha_ref[...] = jnp.broadcast_to(alpha, (B, 128))

    s_proj = dot(static, sp_W_ref[...]) + sp_b_ref[...]
    bias_ref[...] = lc_b1_ref[...] + dot(tp_b_ref[...], W1x_ref[...]) \
        + dot(s_proj, W1c_ref[...])
    M_ref[...] = dot(tp_W_ref[...], W1x_ref[...])


def _mlp_kernel(hA_ref, tf_ref, bias_ref, hB_ref, alpha_ref, M_ref, W1h_ref,
                W2_ref, b2_ref, g_ref, lb_ref, out_ref, hid_ref):
    f32 = jnp.float32
    i = pl.program_id(0)
    nsteps = pl.num_programs(0)

    @pl.when(i < nsteps - 1)
    def _stage_a():
        pre = jnp.dot(hA_ref[...], W1h_ref[...], preferred_element_type=f32)
        pre = pre + jnp.dot(tf_ref[...], M_ref[...],
                            preferred_element_type=f32)
        pre = pre + bias_ref[0]
        hid_ref[i % 2] = _gelu(pre).astype(jnp.bfloat16)

    @pl.when(i > 0)
    def _stage_b():
        hid = hid_ref[(i + 1) % 2]
        dp = jnp.dot(hid, W2_ref[...], preferred_element_type=f32)
        dp = dp + b2_ref[...]
        mu = jnp.mean(dp, axis=-1, keepdims=True)
        var = jnp.mean((dp - mu) ** 2, axis=-1, keepdims=True)
        delta = (dp - mu) / jnp.sqrt(var + 1e-5) * g_ref[...] + lb_ref[...]
        out_ref[...] = hB_ref[...] + alpha_ref[0, 0, 0] * delta


@jax.jit
def kernel(hidden_states, time_features, static_features, bank, phi_W1,
           phi_b1, phi_W2, phi_b2, vp_W, vp_b, rho_W1, rho_b1, rho_W2,
           rho_b2, tp_W, tp_b, sp_W, sp_b, lc_W1, lc_b1, lc_W2, lc_b2,
           ln_g, ln_b):
    f32 = jnp.float32
    row2 = lambda x: x.reshape(1, -1)
    const2 = lambda shape: pl.BlockSpec(shape, lambda i: (0, 0))

    alpha, bias, M = pl.pallas_call(
        _gate_prep_kernel,
        grid=((2 * D_MODEL) // COLS,),
        in_specs=[
            const2((B, N_STATIC)),
            const2((MAX_BANK, N_STATIC)),
            const2((N_STATIC, 2 * KEY_DIM)),
            const2((1, 2 * KEY_DIM)),
            const2((2 * KEY_DIM, KEY_DIM)),
            const2((1, KEY_DIM)),
            const2((N_STATIC, KEY_DIM)),
            const2((1, KEY_DIM)),
            const2((2 * KEY_DIM, KEY_DIM)),
            const2((1, KEY_DIM)),
            const2((KEY_DIM, 1)),
            const2((1, 1)),
            const2((N_TIME, D_MODEL)),
            const2((1, D_MODEL)),
            const2((N_STATIC, D_MODEL)),
            const2((1, D_MODEL)),
            pl.BlockSpec((1, COLS), lambda i: (0, i)),
            pl.BlockSpec((D_MODEL, COLS), lambda i: (1, i)),
            pl.BlockSpec((D_MODEL, COLS), lambda i: (2, i)),
        ],
        out_specs=(
            const2((B, 128)),
            pl.BlockSpec((B, COLS), lambda i: (0, i)),
            pl.BlockSpec((N_TIME, COLS), lambda i: (0, i)),
        ),
        out_shape=(
            jax.ShapeDtypeStruct((B, 128), f32),
            jax.ShapeDtypeStruct((B, 2 * D_MODEL), f32),
            jax.ShapeDtypeStruct((N_TIME, 2 * D_MODEL), f32),
        ),
    )(static_features, bank, phi_W1, row2(phi_b1), phi_W2, row2(phi_b2),
      vp_W, row2(vp_b), rho_W1, row2(rho_b1), rho_W2, row2(rho_b2),
      tp_W, row2(tp_b), sp_W, row2(sp_b), row2(lc_b1), lc_W1, lc_W1)

    h2 = hidden_states.reshape(B * T, D_MODEL)
    tf2 = time_features.reshape(B * T, N_TIME)
    per_batch = T // ROWS
    nblk = B * T // ROWS
    ia = lambda i: jnp.minimum(i, nblk - 1)
    ib = lambda i: jnp.maximum(i - 1, 0)

    out = pl.pallas_call(
        _mlp_kernel,
        grid=(nblk + 1,),
        in_specs=[
            pl.BlockSpec((ROWS, D_MODEL), lambda i: (ia(i), 0)),
            pl.BlockSpec((ROWS, N_TIME), lambda i: (ia(i), 0)),
            pl.BlockSpec((1, 1, 2 * D_MODEL),
                         lambda i: (ia(i) // per_batch, 0, 0)),
            pl.BlockSpec((ROWS, D_MODEL), lambda i: (ib(i), 0)),
            pl.BlockSpec((1, 1, 128), lambda i: (ib(i) // per_batch, 0, 0)),
            pl.BlockSpec((N_TIME, 2 * D_MODEL), lambda i: (0, 0)),
            pl.BlockSpec((D_MODEL, 2 * D_MODEL), lambda i: (0, 0)),
            pl.BlockSpec((2 * D_MODEL, D_MODEL), lambda i: (0, 0)),
            pl.BlockSpec((1, D_MODEL), lambda i: (0, 0)),
            pl.BlockSpec((1, D_MODEL), lambda i: (0, 0)),
            pl.BlockSpec((1, D_MODEL), lambda i: (0, 0)),
        ],
        out_specs=pl.BlockSpec((ROWS, D_MODEL), lambda i: (ib(i), 0)),
        out_shape=jax.ShapeDtypeStruct((B * T, D_MODEL), f32),
        scratch_shapes=[pltpu.VMEM((2, ROWS, 2 * D_MODEL), jnp.bfloat16)],
    )(h2, tf2, bias.reshape(B, 1, 2 * D_MODEL), h2, alpha.reshape(B, 1, 128),
      M, lc_W1, lc_W2, row2(lc_b2), row2(ln_g), row2(ln_b))

    return out.reshape(B, T, D_MODEL)


# speedup vs baseline: 1.0267x; 1.0267x over previous
import jax
import jax.numpy as jnp
from jax.experimental import pallas as pl
from jax.experimental.pallas import tpu as pltpu

D_MODEL = 1024
N_STATIC = 64
N_TIME = 32
KEY_DIM = 128
MAX_BANK = 4096
B = 8
T = 2048

ROWS = 512
COLS = 256


def _gelu(x):
    return 0.5 * x * (1.0 + jax.lax.erf(x * 0.7071067811865476))


def _gate_prep_kernel(static_ref, bank_ref, phi_W1_ref, phi_b1_ref,
                      phi_W2_ref, phi_b2_ref, vp_W_ref, vp_b_ref,
                      rho_W1_ref, rho_b1_ref, rho_W2_ref, rho_b2_ref,
                      tp_W_ref, tp_b_ref, sp_W_ref, sp_b_ref,
                      lc_b1_ref, W1x_ref, W1c_ref,
                      alpha_ref, bias_ref, M_ref):
    static = static_ref[...]
    f32 = jnp.float32

    def dot(a, b):
        return jnp.dot(a, b, preferred_element_type=f32)

    @pl.when(pl.program_id(0) == 0)
    def _attention():
        def phi(x):
            h = _gelu(dot(x, phi_W1_ref[...]) + phi_b1_ref[...])
            return dot(h, phi_W2_ref[...]) + phi_b2_ref[...]

        static_tiled = jnp.broadcast_to(
            static[None],
            (MAX_BANK // B, B, N_STATIC)).reshape(MAX_BANK, N_STATIC)
        rows = jax.lax.broadcasted_iota(jnp.int32, (MAX_BANK, N_STATIC), 0)
        bank_upd = jnp.where(rows < B, static_tiled, bank_ref[...])

        q = phi(static)
        k = phi(bank_upd)
        v = dot(bank_upd, vp_W_ref[...]) + vp_b_ref[...]

        scores = jax.lax.dot_general(
            q, k, (((1,), (1,)), ((), ())),
            preferred_element_type=f32) / (KEY_DIM ** 0.5)
        weights = jax.nn.softmax(scores, axis=-1)
        context = dot(weights, v)

        gate_in = jnp.concatenate([q, context], axis=-1)
        g1 = _gelu(dot(gate_in, rho_W1_ref[...]) + rho_b1_ref[...])
        alpha = jax.nn.sigmoid(dot(g1, rho_W2_ref[...]) + rho_b2_ref[...])
        alpha_ref[...] = jnp.broadcast_to(alpha, (B, 128))

    s_proj = dot(static, sp_W_ref[...]) + sp_b_ref[...]
    bias_ref[...] = lc_b1_ref[...] + dot(tp_b_ref[...], W1x_ref[...]) \
        + dot(s_proj, W1c_ref[...])
    M_ref[...] = dot(tp_W_ref[...], W1x_ref[...])


def _mlp_kernel(hA_ref, tf_ref, bias_ref, hB_ref, alpha_ref, M_ref, W1h_ref,
                W2_ref, b2_ref, g_ref, lb_ref, out_ref, hid_ref):
    f32 = jnp.float32
    i = pl.program_id(0)

    hid = hid_ref[(i + 1) % 2]
    dp = jnp.dot(hid, W2_ref[...], preferred_element_type=f32)
    dp = dp + b2_ref[...]
    mu = jnp.mean(dp, axis=-1, keepdims=True)
    var = jnp.mean((dp - mu) ** 2, axis=-1, keepdims=True)
    delta = (dp - mu) / jnp.sqrt(var + 1e-5) * g_ref[...] + lb_ref[...]
    out_ref[...] = hB_ref[...] + alpha_ref[0, 0, 0] * delta

    pre = jnp.dot(hA_ref[...], W1h_ref[...], preferred_element_type=f32)
    pre = pre + jnp.dot(tf_ref[...], M_ref[...], preferred_element_type=f32)
    pre = pre + bias_ref[0]
    hid_ref[i % 2] = _gelu(pre).astype(jnp.bfloat16)


@jax.jit
def kernel(hidden_states, time_features, static_features, bank, phi_W1,
           phi_b1, phi_W2, phi_b2, vp_W, vp_b, rho_W1, rho_b1, rho_W2,
           rho_b2, tp_W, tp_b, sp_W, sp_b, lc_W1, lc_b1, lc_W2, lc_b2,
           ln_g, ln_b):
    f32 = jnp.float32
    row2 = lambda x: x.reshape(1, -1)
    const2 = lambda shape: pl.BlockSpec(shape, lambda i: (0, 0))

    alpha, bias, M = pl.pallas_call(
        _gate_prep_kernel,
        grid=((2 * D_MODEL) // COLS,),
        in_specs=[
            const2((B, N_STATIC)),
            const2((MAX_BANK, N_STATIC)),
            const2((N_STATIC, 2 * KEY_DIM)),
            const2((1, 2 * KEY_DIM)),
            const2((2 * KEY_DIM, KEY_DIM)),
            const2((1, KEY_DIM)),
            const2((N_STATIC, KEY_DIM)),
            const2((1, KEY_DIM)),
            const2((2 * KEY_DIM, KEY_DIM)),
            const2((1, KEY_DIM)),
            const2((KEY_DIM, 1)),
            const2((1, 1)),
            const2((N_TIME, D_MODEL)),
            const2((1, D_MODEL)),
            const2((N_STATIC, D_MODEL)),
            const2((1, D_MODEL)),
            pl.BlockSpec((1, COLS), lambda i: (0, i)),
            pl.BlockSpec((D_MODEL, COLS), lambda i: (1, i)),
            pl.BlockSpec((D_MODEL, COLS), lambda i: (2, i)),
        ],
        out_specs=(
            const2((B, 128)),
            pl.BlockSpec((B, COLS), lambda i: (0, i)),
            pl.BlockSpec((N_TIME, COLS), lambda i: (0, i)),
        ),
        out_shape=(
            jax.ShapeDtypeStruct((B, 128), f32),
            jax.ShapeDtypeStruct((B, 2 * D_MODEL), f32),
            jax.ShapeDtypeStruct((N_TIME, 2 * D_MODEL), f32),
        ),
    )(static_features, bank, phi_W1, row2(phi_b1), phi_W2, row2(phi_b2),
      vp_W, row2(vp_b), rho_W1, row2(rho_b1), rho_W2, row2(rho_b2),
      tp_W, row2(tp_b), sp_W, row2(sp_b), row2(lc_b1), lc_W1, lc_W1)

    h2 = hidden_states.reshape(B * T, D_MODEL)
    tf2 = time_features.reshape(B * T, N_TIME)
    per_batch = T // ROWS
    nblk = B * T // ROWS
    ia = lambda i: jnp.minimum(i, nblk - 1)
    ib = lambda i: jnp.maximum(i - 1, 0)

    out = pl.pallas_call(
        _mlp_kernel,
        grid=(nblk + 1,),
        in_specs=[
            pl.BlockSpec((ROWS, D_MODEL), lambda i: (ia(i), 0)),
            pl.BlockSpec((ROWS, N_TIME), lambda i: (ia(i), 0)),
            pl.BlockSpec((1, 1, 2 * D_MODEL),
                         lambda i: (ia(i) // per_batch, 0, 0)),
            pl.BlockSpec((ROWS, D_MODEL), lambda i: (ib(i), 0)),
            pl.BlockSpec((1, 1, 128), lambda i: (ib(i) // per_batch, 0, 0)),
            pl.BlockSpec((N_TIME, 2 * D_MODEL), lambda i: (0, 0)),
            pl.BlockSpec((D_MODEL, 2 * D_MODEL), lambda i: (0, 0)),
            pl.BlockSpec((2 * D_MODEL, D_MODEL), lambda i: (0, 0)),
            pl.BlockSpec((1, D_MODEL), lambda i: (0, 0)),
            pl.BlockSpec((1, D_MODEL), lambda i: (0, 0)),
            pl.BlockSpec((1, D_MODEL), lambda i: (0, 0)),
        ],
        out_specs=pl.BlockSpec((ROWS, D_MODEL), lambda i: (ib(i), 0)),
        out_shape=jax.ShapeDtypeStruct((B * T, D_MODEL), f32),
        scratch_shapes=[pltpu.VMEM((2, ROWS, 2 * D_MODEL), jnp.bfloat16)],
    )(h2, tf2, bias.reshape(B, 1, 2 * D_MODEL), h2, alpha.reshape(B, 1, 128),
      M, lc_W1, lc_W2, row2(lc_b2), row2(ln_g), row2(ln_b))

    return out.reshape(B, T, D_MODEL)


# speedup vs baseline: 1.1390x; 1.1094x over previous
import jax
import jax.numpy as jnp
from jax.experimental import pallas as pl
from jax.experimental.pallas import tpu as pltpu

D_MODEL = 1024
N_STATIC = 64
N_TIME = 32
KEY_DIM = 128
MAX_BANK = 4096
B = 8
T = 2048

ROWS = 1024
COLS = 256


def _gelu(x):
    return 0.5 * x * (1.0 + jax.lax.erf(x * 0.7071067811865476))


def _gate_prep_kernel(static_ref, bank_ref, phi_W1_ref, phi_b1_ref,
                      phi_W2_ref, phi_b2_ref, vp_W_ref, vp_b_ref,
                      rho_W1_ref, rho_b1_ref, rho_W2_ref, rho_b2_ref,
                      tp_W_ref, tp_b_ref, sp_W_ref, sp_b_ref,
                      lc_b1_ref, W1x_ref, W1c_ref,
                      alpha_ref, bias_ref, M_ref):
    static = static_ref[...]
    f32 = jnp.float32

    def dot(a, b):
        return jnp.dot(a, b, preferred_element_type=f32)

    @pl.when(pl.program_id(0) == 0)
    def _attention():
        def phi(x):
            h = _gelu(dot(x, phi_W1_ref[...]) + phi_b1_ref[...])
            return dot(h, phi_W2_ref[...]) + phi_b2_ref[...]

        static_tiled = jnp.broadcast_to(
            static[None],
            (MAX_BANK // B, B, N_STATIC)).reshape(MAX_BANK, N_STATIC)
        rows = jax.lax.broadcasted_iota(jnp.int32, (MAX_BANK, N_STATIC), 0)
        bank_upd = jnp.where(rows < B, static_tiled, bank_ref[...])

        q = phi(static)
        k = phi(bank_upd)
        v = dot(bank_upd, vp_W_ref[...]) + vp_b_ref[...]

        scores = jax.lax.dot_general(
            q, k, (((1,), (1,)), ((), ())),
            preferred_element_type=f32) / (KEY_DIM ** 0.5)
        weights = jax.nn.softmax(scores, axis=-1)
        context = dot(weights, v)

        gate_in = jnp.concatenate([q, context], axis=-1)
        g1 = _gelu(dot(gate_in, rho_W1_ref[...]) + rho_b1_ref[...])
        alpha = jax.nn.sigmoid(dot(g1, rho_W2_ref[...]) + rho_b2_ref[...])
        alpha_ref[...] = jnp.broadcast_to(alpha, (B, 128))

    s_proj = dot(static, sp_W_ref[...]) + sp_b_ref[...]
    bias_ref[...] = lc_b1_ref[...] + dot(tp_b_ref[...], W1x_ref[...]) \
        + dot(s_proj, W1c_ref[...])
    M_ref[...] = dot(tp_W_ref[...], W1x_ref[...])


def _mlp_kernel(h_ref, tf_ref, bias_ref, alpha_ref, M_ref, W1h_ref, W2_ref,
                b2_ref, g_ref, lb_ref, out_ref):
    f32 = jnp.float32
    bf16 = jnp.bfloat16
    h = h_ref[...]
    tf = tf_ref[...]
    D2 = D_MODEL
    dp = None
    for a, b in ((0, D2), (D2, 2 * D2)):
        pre = jnp.dot(h, W1h_ref[:, a:b], preferred_element_type=f32)
        pre = pre + jnp.dot(tf, M_ref[:, a:b], preferred_element_type=f32)
        pre = pre + bias_ref[0][:, a:b]
        hid = _gelu(pre).astype(bf16)
        c = jnp.dot(hid, W2_ref[a:b, :], preferred_element_type=f32)
        dp = c if dp is None else dp + c
    dp = dp + b2_ref[...]
    mu = jnp.mean(dp, axis=-1, keepdims=True)
    var = jnp.mean((dp - mu) ** 2, axis=-1, keepdims=True)
    delta = (dp - mu) / jnp.sqrt(var + 1e-5) * g_ref[...] + lb_ref[...]
    out_ref[...] = h + alpha_ref[0, 0, 0] * delta


@jax.jit
def kernel(hidden_states, time_features, static_features, bank, phi_W1,
           phi_b1, phi_W2, phi_b2, vp_W, vp_b, rho_W1, rho_b1, rho_W2,
           rho_b2, tp_W, tp_b, sp_W, sp_b, lc_W1, lc_b1, lc_W2, lc_b2,
           ln_g, ln_b):
    f32 = jnp.float32
    row2 = lambda x: x.reshape(1, -1)
    const2 = lambda shape: pl.BlockSpec(shape, lambda i: (0, 0))

    alpha, bias, M = pl.pallas_call(
        _gate_prep_kernel,
        grid=((2 * D_MODEL) // COLS,),
        in_specs=[
            const2((B, N_STATIC)),
            const2((MAX_BANK, N_STATIC)),
            const2((N_STATIC, 2 * KEY_DIM)),
            const2((1, 2 * KEY_DIM)),
            const2((2 * KEY_DIM, KEY_DIM)),
            const2((1, KEY_DIM)),
            const2((N_STATIC, KEY_DIM)),
            const2((1, KEY_DIM)),
            const2((2 * KEY_DIM, KEY_DIM)),
            const2((1, KEY_DIM)),
            const2((KEY_DIM, 1)),
            const2((1, 1)),
            const2((N_TIME, D_MODEL)),
            const2((1, D_MODEL)),
            const2((N_STATIC, D_MODEL)),
            const2((1, D_MODEL)),
            pl.BlockSpec((1, COLS), lambda i: (0, i)),
            pl.BlockSpec((D_MODEL, COLS), lambda i: (1, i)),
            pl.BlockSpec((D_MODEL, COLS), lambda i: (2, i)),
        ],
        out_specs=(
            const2((B, 128)),
            pl.BlockSpec((B, COLS), lambda i: (0, i)),
            pl.BlockSpec((N_TIME, COLS), lambda i: (0, i)),
        ),
        out_shape=(
            jax.ShapeDtypeStruct((B, 128), f32),
            jax.ShapeDtypeStruct((B, 2 * D_MODEL), f32),
            jax.ShapeDtypeStruct((N_TIME, 2 * D_MODEL), f32),
        ),
    )(static_features, bank, phi_W1, row2(phi_b1), phi_W2, row2(phi_b2),
      vp_W, row2(vp_b), rho_W1, row2(rho_b1), rho_W2, row2(rho_b2),
      tp_W, row2(tp_b), sp_W, row2(sp_b), row2(lc_b1), lc_W1, lc_W1)

    h2 = hidden_states.reshape(B * T, D_MODEL)
    tf2 = time_features.reshape(B * T, N_TIME)
    per_batch = T // ROWS
    grid = (B * T // ROWS,)

    out = pl.pallas_call(
        _mlp_kernel,
        grid=grid,
        in_specs=[
            pl.BlockSpec((ROWS, D_MODEL), lambda i: (i, 0)),
            pl.BlockSpec((ROWS, N_TIME), lambda i: (i, 0)),
            pl.BlockSpec((1, 1, 2 * D_MODEL), lambda i: (i // per_batch, 0, 0)),
            pl.BlockSpec((1, 1, 128), lambda i: (i // per_batch, 0, 0)),
            pl.BlockSpec((N_TIME, 2 * D_MODEL), lambda i: (0, 0)),
            pl.BlockSpec((D_MODEL, 2 * D_MODEL), lambda i: (0, 0)),
            pl.BlockSpec((2 * D_MODEL, D_MODEL), lambda i: (0, 0)),
            pl.BlockSpec((1, D_MODEL), lambda i: (0, 0)),
            pl.BlockSpec((1, D_MODEL), lambda i: (0, 0)),
            pl.BlockSpec((1, D_MODEL), lambda i: (0, 0)),
        ],
        out_specs=pl.BlockSpec((ROWS, D_MODEL), lambda i: (i, 0)),
        out_shape=jax.ShapeDtypeStruct((B * T, D_MODEL), f32),
        compiler_params=pltpu.CompilerParams(
            dimension_semantics=("parallel",)),
    )(h2, tf2, bias.reshape(B, 1, 2 * D_MODEL), alpha.reshape(B, 1, 128),
      M, lc_W1, lc_W2, row2(lc_b2), row2(ln_g), row2(ln_b))

    return out.reshape(B, T, D_MODEL)


# speedup vs baseline: 1.1501x; 1.0098x over previous
import jax
import jax.numpy as jnp
from jax.experimental import pallas as pl
from jax.experimental.pallas import tpu as pltpu

D_MODEL = 1024
N_STATIC = 64
N_TIME = 32
KEY_DIM = 128
MAX_BANK = 4096
B = 8
T = 2048

ROWS = 1024
COLS = 256


def _gelu(x):
    return 0.5 * x * (1.0 + jax.lax.erf(x * 0.7071067811865476))


def _gate_prep_kernel(static_ref, bank_ref, phi_W1_ref, phi_b1_ref,
                      phi_W2_ref, phi_b2_ref, vp_W_ref, vp_b_ref,
                      rho_W1_ref, rho_b1_ref, rho_W2_ref, rho_b2_ref,
                      tp_W_ref, tp_b_ref, sp_W_ref, sp_b_ref,
                      lc_b1_ref, W1x_ref, W1c_ref,
                      alpha_ref, bias_ref, M_ref):
    static = static_ref[...]
    f32 = jnp.float32

    def dot(a, b):
        return jnp.dot(a, b, preferred_element_type=f32)

    @pl.when(pl.program_id(0) == 0)
    def _attention():
        def phi(x):
            h = _gelu(dot(x, phi_W1_ref[...]) + phi_b1_ref[...])
            return dot(h, phi_W2_ref[...]) + phi_b2_ref[...]

        static_tiled = jnp.broadcast_to(
            static[None],
            (MAX_BANK // B, B, N_STATIC)).reshape(MAX_BANK, N_STATIC)
        rows = jax.lax.broadcasted_iota(jnp.int32, (MAX_BANK, N_STATIC), 0)
        bank_upd = jnp.where(rows < B, static_tiled, bank_ref[...])

        q = phi(static)
        k = phi(bank_upd)
        v = dot(bank_upd, vp_W_ref[...]) + vp_b_ref[...]

        scores = jax.lax.dot_general(
            q, k, (((1,), (1,)), ((), ())),
            preferred_element_type=f32) / (KEY_DIM ** 0.5)
        weights = jax.nn.softmax(scores, axis=-1)
        context = dot(weights, v)

        gate_in = jnp.concatenate([q, context], axis=-1)
        g1 = _gelu(dot(gate_in, rho_W1_ref[...]) + rho_b1_ref[...])
        alpha = jax.nn.sigmoid(dot(g1, rho_W2_ref[...]) + rho_b2_ref[...])
        alpha_ref[...] = jnp.broadcast_to(alpha, (B, 128))

    s_proj = dot(static, sp_W_ref[...]) + sp_b_ref[...]
    bias_ref[...] = lc_b1_ref[...] + dot(tp_b_ref[...], W1x_ref[...]) \
        + dot(s_proj, W1c_ref[...])
    M_ref[...] = dot(tp_W_ref[...], W1x_ref[...])


def _mlp_kernel(h_ref, tf_ref, bias_ref, alpha_ref, M_ref, W1h_ref, W2_ref,
                b2_ref, g_ref, lb_ref, out_ref):
    f32 = jnp.float32
    h = h_ref[...]
    pre = jnp.dot(h, W1h_ref[...], preferred_element_type=f32)
    pre = pre + jnp.dot(tf_ref[...], M_ref[...], preferred_element_type=f32)
    pre = pre + bias_ref[0]
    hid = _gelu(pre).astype(jnp.bfloat16)
    dp = jnp.dot(hid, W2_ref[...], preferred_element_type=f32) + b2_ref[...]
    mu = jnp.mean(dp, axis=-1, keepdims=True)
    var = jnp.mean((dp - mu) ** 2, axis=-1, keepdims=True)
    delta = (dp - mu) / jnp.sqrt(var + 1e-5) * g_ref[...] + lb_ref[...]
    out_ref[...] = h + alpha_ref[0, 0, 0] * delta


@jax.jit
def kernel(hidden_states, time_features, static_features, bank, phi_W1,
           phi_b1, phi_W2, phi_b2, vp_W, vp_b, rho_W1, rho_b1, rho_W2,
           rho_b2, tp_W, tp_b, sp_W, sp_b, lc_W1, lc_b1, lc_W2, lc_b2,
           ln_g, ln_b):
    f32 = jnp.float32
    row2 = lambda x: x.reshape(1, -1)
    const2 = lambda shape: pl.BlockSpec(shape, lambda i: (0, 0))

    alpha, bias, M = pl.pallas_call(
        _gate_prep_kernel,
        grid=((2 * D_MODEL) // COLS,),
        in_specs=[
            const2((B, N_STATIC)),
            const2((MAX_BANK, N_STATIC)),
            const2((N_STATIC, 2 * KEY_DIM)),
            const2((1, 2 * KEY_DIM)),
            const2((2 * KEY_DIM, KEY_DIM)),
            const2((1, KEY_DIM)),
            const2((N_STATIC, KEY_DIM)),
            const2((1, KEY_DIM)),
            const2((2 * KEY_DIM, KEY_DIM)),
            const2((1, KEY_DIM)),
            const2((KEY_DIM, 1)),
            const2((1, 1)),
            const2((N_TIME, D_MODEL)),
            const2((1, D_MODEL)),
            const2((N_STATIC, D_MODEL)),
            const2((1, D_MODEL)),
            pl.BlockSpec((1, COLS), lambda i: (0, i)),
            pl.BlockSpec((D_MODEL, COLS), lambda i: (1, i)),
            pl.BlockSpec((D_MODEL, COLS), lambda i: (2, i)),
        ],
        out_specs=(
            const2((B, 128)),
            pl.BlockSpec((B, COLS), lambda i: (0, i)),
            pl.BlockSpec((N_TIME, COLS), lambda i: (0, i)),
        ),
        out_shape=(
            jax.ShapeDtypeStruct((B, 128), f32),
            jax.ShapeDtypeStruct((B, 2 * D_MODEL), f32),
            jax.ShapeDtypeStruct((N_TIME, 2 * D_MODEL), f32),
        ),
    )(static_features, bank, phi_W1, row2(phi_b1), phi_W2, row2(phi_b2),
      vp_W, row2(vp_b), rho_W1, row2(rho_b1), rho_W2, row2(rho_b2),
      tp_W, row2(tp_b), sp_W, row2(sp_b), row2(lc_b1), lc_W1, lc_W1)

    h2 = hidden_states.reshape(B * T, D_MODEL)
    tf2 = time_features.reshape(B * T, N_TIME)
    per_batch = T // ROWS
    grid = (B * T // ROWS,)

    out = pl.pallas_call(
        _mlp_kernel,
        grid=grid,
        in_specs=[
            pl.BlockSpec((ROWS, D_MODEL), lambda i: (i, 0)),
            pl.BlockSpec((ROWS, N_TIME), lambda i: (i, 0)),
            pl.BlockSpec((1, 1, 2 * D_MODEL), lambda i: (i // per_batch, 0, 0)),
            pl.BlockSpec((1, 1, 128), lambda i: (i // per_batch, 0, 0)),
            pl.BlockSpec((N_TIME, 2 * D_MODEL), lambda i: (0, 0)),
            pl.BlockSpec((D_MODEL, 2 * D_MODEL), lambda i: (0, 0)),
            pl.BlockSpec((2 * D_MODEL, D_MODEL), lambda i: (0, 0)),
            pl.BlockSpec((1, D_MODEL), lambda i: (0, 0)),
            pl.BlockSpec((1, D_MODEL), lambda i: (0, 0)),
            pl.BlockSpec((1, D_MODEL), lambda i: (0, 0)),
        ],
        out_specs=pl.BlockSpec((ROWS, D_MODEL), lambda i: (i, 0)),
        out_shape=jax.ShapeDtypeStruct((B * T, D_MODEL), f32),
    )(h2, tf2, bias.reshape(B, 1, 2 * D_MODEL), alpha.reshape(B, 1, 128),
      M, lc_W1, lc_W2, row2(lc_b2), row2(ln_g), row2(ln_b))

    return out.reshape(B, T, D_MODEL)


# speedup vs baseline: 1.1505x; 1.0003x over previous
import jax
import jax.numpy as jnp
from jax.experimental import pallas as pl

D_MODEL = 1024
N_STATIC = 64
N_TIME = 32
KEY_DIM = 128
MAX_BANK = 4096
B = 8
T = 2048

ROWS = 1024
COLS = 256


def _gelu(x):
    return 0.5 * x * (1.0 + jax.lax.erf(x * 0.7071067811865476))


def _gate_prep_kernel(static_ref, bank_ref, phi_W1_ref, phi_b1_ref,
                      phi_W2_ref, phi_b2_ref, vp_W_ref, vp_b_ref,
                      rho_W1_ref, rho_b1_ref, rho_W2_ref, rho_b2_ref,
                      tp_W_ref, tp_b_ref, sp_W_ref, sp_b_ref,
                      lc_b1_ref, W1x_ref, W1c_ref,
                      alpha_ref, bias_ref, M_ref):
    static = static_ref[...]
    f32 = jnp.float32

    def dot(a, b):
        return jnp.dot(a, b, preferred_element_type=f32)

    @pl.when(pl.program_id(0) == 0)
    def _attention():
        def phi(x):
            h = _gelu(dot(x, phi_W1_ref[...]) + phi_b1_ref[...])
            return dot(h, phi_W2_ref[...]) + phi_b2_ref[...]

        static_tiled = jnp.broadcast_to(
            static[None],
            (MAX_BANK // B, B, N_STATIC)).reshape(MAX_BANK, N_STATIC)
        rows = jax.lax.broadcasted_iota(jnp.int32, (MAX_BANK, N_STATIC), 0)
        bank_upd = jnp.where(rows < B, static_tiled, bank_ref[...])

        q = phi(static)
        k = phi(bank_upd)
        v = dot(bank_upd, vp_W_ref[...]) + vp_b_ref[...]

        scores = jax.lax.dot_general(
            q, k, (((1,), (1,)), ((), ())),
            preferred_element_type=f32) / (KEY_DIM ** 0.5)
        weights = jax.nn.softmax(scores, axis=-1)
        context = dot(weights, v)

        gate_in = jnp.concatenate([q, context], axis=-1)
        g1 = _gelu(dot(gate_in, rho_W1_ref[...]) + rho_b1_ref[...])
        alpha = jax.nn.sigmoid(dot(g1, rho_W2_ref[...]) + rho_b2_ref[...])
        alpha_ref[...] = jnp.broadcast_to(alpha, (B, 128))

    s_proj = dot(static, sp_W_ref[...]) + sp_b_ref[...]
    bias_ref[...] = lc_b1_ref[...] + dot(tp_b_ref[...], W1x_ref[...]) \
        + dot(s_proj, W1c_ref[...])
    M_ref[...] = dot(tp_W_ref[...], W1x_ref[...])


def _mlp_kernel(h_ref, tf_ref, bias_ref, alpha_ref, M_ref, W1h_ref, W2_ref,
                b2_ref, g_ref, lb_ref, out_ref):
    f32 = jnp.float32
    h = h_ref[...]
    pre = jnp.dot(h, W1h_ref[...], preferred_element_type=f32)
    pre = pre + jnp.dot(tf_ref[...], M_ref[...], preferred_element_type=f32)
    pre = pre + bias_ref[0]
    hid = _gelu(pre).astype(jnp.bfloat16)
    dp = jnp.dot(hid, W2_ref[...], preferred_element_type=f32) + b2_ref[...]
    mu = jnp.mean(dp, axis=-1, keepdims=True)
    var = jnp.mean((dp - mu) ** 2, axis=-1, keepdims=True)
    delta = (dp - mu) / jnp.sqrt(var + 1e-5) * g_ref[...] + lb_ref[...]
    out_ref[...] = h + alpha_ref[0, 0, 0] * delta


@jax.jit
def kernel(hidden_states, time_features, static_features, bank, phi_W1,
           phi_b1, phi_W2, phi_b2, vp_W, vp_b, rho_W1, rho_b1, rho_W2,
           rho_b2, tp_W, tp_b, sp_W, sp_b, lc_W1, lc_b1, lc_W2, lc_b2,
           ln_g, ln_b):
    f32 = jnp.float32
    row2 = lambda x: x.reshape(1, -1)
    const2 = lambda shape: pl.BlockSpec(shape, lambda i: (0, 0))

    alpha, bias, M = pl.pallas_call(
        _gate_prep_kernel,
        grid=((2 * D_MODEL) // COLS,),
        in_specs=[
            const2((B, N_STATIC)),
            const2((MAX_BANK, N_STATIC)),
            const2((N_STATIC, 2 * KEY_DIM)),
            const2((1, 2 * KEY_DIM)),
            const2((2 * KEY_DIM, KEY_DIM)),
            const2((1, KEY_DIM)),
            const2((N_STATIC, KEY_DIM)),
            const2((1, KEY_DIM)),
            const2((2 * KEY_DIM, KEY_DIM)),
            const2((1, KEY_DIM)),
            const2((KEY_DIM, 1)),
            const2((1, 1)),
            const2((N_TIME, D_MODEL)),
            const2((1, D_MODEL)),
            const2((N_STATIC, D_MODEL)),
            const2((1, D_MODEL)),
            pl.BlockSpec((1, COLS), lambda i: (0, i)),
            pl.BlockSpec((D_MODEL, COLS), lambda i: (1, i)),
            pl.BlockSpec((D_MODEL, COLS), lambda i: (2, i)),
        ],
        out_specs=(
            const2((B, 128)),
            pl.BlockSpec((B, COLS), lambda i: (0, i)),
            pl.BlockSpec((N_TIME, COLS), lambda i: (0, i)),
        ),
        out_shape=(
            jax.ShapeDtypeStruct((B, 128), f32),
            jax.ShapeDtypeStruct((B, 2 * D_MODEL), f32),
            jax.ShapeDtypeStruct((N_TIME, 2 * D_MODEL), f32),
        ),
    )(static_features, bank, phi_W1, row2(phi_b1), phi_W2, row2(phi_b2),
      vp_W, row2(vp_b), rho_W1, row2(rho_b1), rho_W2, row2(rho_b2),
      tp_W, row2(tp_b), sp_W, row2(sp_b), row2(lc_b1), lc_W1, lc_W1)

    h2 = hidden_states.reshape(B * T, D_MODEL)
    tf2 = time_features.reshape(B * T, N_TIME)
    per_batch = T // ROWS
    grid = (B * T // ROWS,)

    out = pl.pallas_call(
        _mlp_kernel,
        grid=grid,
        in_specs=[
            pl.BlockSpec((ROWS, D_MODEL), lambda i: (i, 0)),
            pl.BlockSpec((ROWS, N_TIME), lambda i: (i, 0)),
            pl.BlockSpec((1, 1, 2 * D_MODEL), lambda i: (i // per_batch, 0, 0)),
            pl.BlockSpec((1, 1, 128), lambda i: (i // per_batch, 0, 0)),
            pl.BlockSpec((N_TIME, 2 * D_MODEL), lambda i: (0, 0)),
            pl.BlockSpec((D_MODEL, 2 * D_MODEL), lambda i: (0, 0)),
            pl.BlockSpec((2 * D_MODEL, D_MODEL), lambda i: (0, 0)),
            pl.BlockSpec((1, D_MODEL), lambda i: (0, 0)),
            pl.BlockSpec((1, D_MODEL), lambda i: (0, 0)),
            pl.BlockSpec((1, D_MODEL), lambda i: (0, 0)),
        ],
        out_specs=pl.BlockSpec((ROWS, D_MODEL), lambda i: (i, 0)),
        out_shape=jax.ShapeDtypeStruct((B * T, D_MODEL), f32),
    )(h2, tf2, bias.reshape(B, 1, 2 * D_MODEL), alpha.reshape(B, 1, 128),
      M, lc_W1, lc_W2, row2(lc_b2), row2(ln_g), row2(ln_b))

    return out.reshape(B, T, D_MODEL)
